# baseline (speedup 1.0000x reference)
import functools
import math

import jax
import jax.numpy as jnp
from jax.experimental import pallas as pl
from jax.experimental.pallas import tpu as pltpu

_EPS = 1e-6
_NEG = -1e30


def _dot_t(a, b):
    return jax.lax.dot_general(a, b, (((1,), (1,)), ((), ())),
                               preferred_element_type=jnp.float32)


def _dot3(a, b):
    return jnp.dot(a, b, preferred_element_type=jnp.float32)


def _rms(xf, w):
    ms = jnp.mean(xf * xf, axis=-1, keepdims=True)
    return xf / jnp.sqrt(ms + _EPS) * w


def _qkv_body(x_ref, nw_ref, w_ref, o_ref):
    xn = _rms(x_ref[...], nw_ref[...])
    o_ref[...] = _dot_t(xn, w_ref[...])


def _attn_body(qkv_ref, o_ref, *, bq, hd, nh, d):
    i = pl.program_id(0)
    tri = (jax.lax.broadcasted_iota(jnp.int32, (bq, bq), 1)
           <= jax.lax.broadcasted_iota(jnp.int32, (bq, bq), 0))
    ones_blk = jnp.ones((bq, 128 - hd), jnp.float32)
    inv = 1.0 / math.sqrt(hd)
    for h in range(nh):
        q = qkv_ref[pl.ds(i * bq, bq), h * hd:(h + 1) * hd]

        def step(j, m, acc, masked):
            k = qkv_ref[pl.ds(j * bq, bq), d + h * hd:d + (h + 1) * hd]
            v = qkv_ref[pl.ds(j * bq, bq), 2 * d + h * hd:2 * d + (h + 1) * hd]
            s = _dot_t(q, k) * inv
            if masked:
                s = jnp.where(tri, s, _NEG)
            mj = jnp.max(s, axis=-1, keepdims=True)
            m_new = jnp.maximum(m, mj)
            p = jnp.exp(s - m_new)
            scale = jnp.exp(m - m_new)
            v2 = jnp.concatenate([v, ones_blk], axis=1)
            return m_new, acc * scale + _dot3(p, v2)

        m0 = jnp.full((bq, 1), _NEG, jnp.float32)
        a0 = jnp.zeros((bq, 128), jnp.float32)
        m, acc = jax.lax.fori_loop(
            0, i, lambda j, c: step(j, c[0], c[1], False), (m0, a0))
        m, acc = step(i, m, acc, True)
        o_ref[:, h * hd:(h + 1) * hd] = acc[:, :hd] / acc[:, hd:hd + 1]


def _proj_body(attn_ref, x_ref, ow_ref, nw_ref, wsg_ref, wsu_ref, wsd_ref,
               xres_ref, xn2_ref, shared_ref):
    a = _dot_t(attn_ref[...], ow_ref[...])
    xr = x_ref[...] + a
    xres_ref[...] = xr
    xn = _rms(xr, nw_ref[...])
    xnb = xn.astype(jnp.bfloat16)
    xn2_ref[...] = xnb
    g = jnp.dot(xnb, wsg_ref[...].astype(jnp.bfloat16),
                preferred_element_type=jnp.float32)
    u = jnp.dot(xnb, wsu_ref[...].astype(jnp.bfloat16),
                preferred_element_type=jnp.float32)
    hs = (g * jax.nn.sigmoid(g) * u).astype(jnp.bfloat16)
    shared_ref[...] = jnp.dot(hs, wsd_ref[...].astype(jnp.bfloat16),
                              preferred_element_type=jnp.float32)


def _router_body(xres_ref, nw_ref, rwt_ref, cw_ref, aux_ref, *, ne, coeff):
    xn = _rms(xres_ref[...], nw_ref[...])
    logits = _dot_t(xn, rwt_ref[...])
    lm = jnp.max(logits, axis=-1, keepdims=True)
    ex = jnp.exp(logits - lm)
    probs = ex / jnp.sum(ex, axis=-1, keepdims=True)
    idx = jax.lax.broadcasted_iota(jnp.int32, probs.shape, 1)
    m1 = jnp.max(probs, axis=-1, keepdims=True)
    i1 = jnp.min(jnp.where(probs == m1, idx, ne), axis=-1, keepdims=True)
    oh1 = (idx == i1)
    p2 = jnp.where(oh1, _NEG, probs)
    m2 = jnp.max(p2, axis=-1, keepdims=True)
    i2 = jnp.min(jnp.where(p2 == m2, idx, ne), axis=-1, keepdims=True)
    oh2 = (idx == i2)
    denom = m1 + m2
    cw_ref[...] = (jnp.where(oh1, m1, 0.0) + jnp.where(oh2, m2, 0.0)) / denom
    frac = jnp.mean((oh1 | oh2).astype(jnp.float32), axis=0, keepdims=True)
    pmean = jnp.mean(probs, axis=0, keepdims=True)
    aux_ref[...] = jnp.sum(frac * pmean).reshape(1, 1) * (coeff * ne)


def _moe_body(xn2_ref, wg_ref, wu_ref, wd_ref, cw_ref, xres_ref, shared_ref,
              o_ref, *, ne):
    e = pl.program_id(0)
    xb = xn2_ref[...]
    g = jnp.dot(xb, wg_ref[0].astype(jnp.bfloat16),
                preferred_element_type=jnp.float32)
    u = jnp.dot(xb, wu_ref[0].astype(jnp.bfloat16),
                preferred_element_type=jnp.float32)
    hh = (g * jax.nn.sigmoid(g) * u).astype(jnp.bfloat16)
    ye = jnp.dot(hh, wd_ref[0].astype(jnp.bfloat16),
                 preferred_element_type=jnp.float32)
    cwb = cw_ref[...]
    lane = jax.lax.broadcasted_iota(jnp.int32, cwb.shape, 1)
    w_col = jnp.sum(jnp.where(lane == e, cwb, 0.0), axis=-1, keepdims=True)
    contrib = w_col * ye

    @pl.when(e == 0)
    def _():
        o_ref[...] = xres_ref[...] + shared_ref[...] + contrib

    @pl.when(e > 0)
    def _():
        o_ref[...] += contrib


def kernel(x, attn_norm_w, qkv_w, out_w, ffn_norm_w, router_w, w_gate, w_up,
           w_down, ws_gate, ws_up, ws_down):
    B, T, D = x.shape
    E, _, F = w_gate.shape
    H = 16
    hd = D // H
    BT = min(512, T)
    BM = min(512, T)
    x2 = x.reshape(T, D)
    bf = jnp.bfloat16

    anw = attn_norm_w.reshape(1, D)
    fnw = ffn_norm_w.reshape(1, D)

    qkv = pl.pallas_call(
        _qkv_body,
        grid=(T // BT,),
        in_specs=[
            pl.BlockSpec((BT, D), lambda i: (i, 0)),
            pl.BlockSpec((1, D), lambda i: (0, 0)),
            pl.BlockSpec((3 * D, D), lambda i: (0, 0)),
        ],
        out_specs=pl.BlockSpec((BT, 3 * D), lambda i: (i, 0)),
        out_shape=jax.ShapeDtypeStruct((T, 3 * D), jnp.float32),
        compiler_params=pltpu.CompilerParams(
            dimension_semantics=("arbitrary",)),
    )(x2, anw, qkv_w)

    BQ = min(512, T)
    attn = pl.pallas_call(
        functools.partial(_attn_body, bq=BQ, hd=hd, nh=H, d=D),
        grid=(T // BQ,),
        in_specs=[
            pl.BlockSpec((T, 3 * D), lambda i: (0, 0)),
        ],
        out_specs=pl.BlockSpec((BQ, D), lambda i: (i, 0)),
        out_shape=jax.ShapeDtypeStruct((T, D), jnp.float32),
        compiler_params=pltpu.CompilerParams(
            dimension_semantics=("arbitrary",)),
    )(qkv)

    xres, xn2, shared = pl.pallas_call(
        _proj_body,
        grid=(T // BT,),
        in_specs=[
            pl.BlockSpec((BT, D), lambda i: (i, 0)),
            pl.BlockSpec((BT, D), lambda i: (i, 0)),
            pl.BlockSpec((D, D), lambda i: (0, 0)),
            pl.BlockSpec((1, D), lambda i: (0, 0)),
            pl.BlockSpec((D, F), lambda i: (0, 0)),
            pl.BlockSpec((D, F), lambda i: (0, 0)),
            pl.BlockSpec((F, D), lambda i: (0, 0)),
        ],
        out_specs=[
            pl.BlockSpec((BT, D), lambda i: (i, 0)),
            pl.BlockSpec((BT, D), lambda i: (i, 0)),
            pl.BlockSpec((BT, D), lambda i: (i, 0)),
        ],
        out_shape=[
            jax.ShapeDtypeStruct((T, D), jnp.float32),
            jax.ShapeDtypeStruct((T, D), bf),
            jax.ShapeDtypeStruct((T, D), jnp.float32),
        ],
        compiler_params=pltpu.CompilerParams(
            dimension_semantics=("arbitrary",)),
    )(attn, x2, out_w, fnw, ws_gate, ws_up, ws_down)

    cw, aux = pl.pallas_call(
        functools.partial(_router_body, ne=E, coeff=0.01),
        grid=(1,),
        in_specs=[
            pl.BlockSpec((T, D), lambda i: (0, 0)),
            pl.BlockSpec((1, D), lambda i: (0, 0)),
            pl.BlockSpec((E, D), lambda i: (0, 0)),
        ],
        out_specs=[
            pl.BlockSpec((T, E), lambda i: (0, 0)),
            pl.BlockSpec((1, 1), lambda i: (0, 0)),
        ],
        out_shape=[
            jax.ShapeDtypeStruct((T, E), jnp.float32),
            jax.ShapeDtypeStruct((1, 1), jnp.float32),
        ],
    )(xres, fnw, router_w)

    y = pl.pallas_call(
        functools.partial(_moe_body, ne=E),
        grid=(E,),
        in_specs=[
            pl.BlockSpec((T, D), lambda e: (0, 0)),
            pl.BlockSpec((1, D, F), lambda e: (e, 0, 0)),
            pl.BlockSpec((1, D, F), lambda e: (e, 0, 0)),
            pl.BlockSpec((1, F, D), lambda e: (e, 0, 0)),
            pl.BlockSpec((T, E), lambda e: (0, 0)),
            pl.BlockSpec((T, D), lambda e: (0, 0)),
            pl.BlockSpec((T, D), lambda e: (0, 0)),
        ],
        out_specs=pl.BlockSpec((T, D), lambda e: (0, 0)),
        out_shape=jax.ShapeDtypeStruct((T, D), jnp.float32),
        compiler_params=pltpu.CompilerParams(
            dimension_semantics=("arbitrary",)),
    )(xn2, w_gate, w_up, w_down, cw, xres, shared)

    return (y.reshape(B, T, D), aux[0, 0])


# speedup vs baseline: 1.0351x; 1.0351x over previous
import functools
import math

import jax
import jax.numpy as jnp
from jax.experimental import pallas as pl
from jax.experimental.pallas import tpu as pltpu

_EPS = 1e-6
_NEG = -1e30


def _dot_t(a, b):
    return jax.lax.dot_general(a, b, (((1,), (1,)), ((), ())),
                               preferred_element_type=jnp.float32)


def _dot3(a, b):
    return jnp.dot(a, b, preferred_element_type=jnp.float32)


def _rms(xf, w):
    ms = jnp.mean(xf * xf, axis=-1, keepdims=True)
    return xf / jnp.sqrt(ms + _EPS) * w


def _qkv_body(x_ref, nw_ref, w_ref, o_ref):
    xn = _rms(x_ref[...], nw_ref[...])
    o_ref[...] = _dot_t(xn, w_ref[...])


def _attn_body(qkv_ref, o_ref, *, bq, hd, nh, d):
    i = pl.program_id(0)
    tri = (jax.lax.broadcasted_iota(jnp.int32, (bq, bq), 1)
           <= jax.lax.broadcasted_iota(jnp.int32, (bq, bq), 0))
    ones_blk = jnp.ones((bq, 128 - hd), jnp.float32)
    inv = 1.0 / math.sqrt(hd)
    for h in range(nh):
        q = qkv_ref[pl.ds(i * bq, bq), h * hd:(h + 1) * hd]

        def step(j, m, acc, masked):
            k = qkv_ref[pl.ds(j * bq, bq), d + h * hd:d + (h + 1) * hd]
            v = qkv_ref[pl.ds(j * bq, bq), 2 * d + h * hd:2 * d + (h + 1) * hd]
            s = _dot_t(q, k) * inv
            if masked:
                s = jnp.where(tri, s, _NEG)
            mj = jnp.max(s, axis=-1, keepdims=True)
            m_new = jnp.maximum(m, mj)
            p = jnp.exp(s - m_new)
            scale = jnp.exp(m - m_new)
            v2 = jnp.concatenate([v, ones_blk], axis=1)
            return m_new, acc * scale + _dot3(p, v2)

        m0 = jnp.full((bq, 1), _NEG, jnp.float32)
        a0 = jnp.zeros((bq, 128), jnp.float32)
        m, acc = jax.lax.fori_loop(
            0, i, lambda j, c: step(j, c[0], c[1], False), (m0, a0))
        m, acc = step(i, m, acc, True)
        o_ref[:, h * hd:(h + 1) * hd] = acc[:, :hd] / acc[:, hd:hd + 1]


def _proj_body(attn_ref, x_ref, ow_ref, nw_ref, wsg_ref, wsu_ref, wsd_ref,
               xres_ref, xn2_ref, shared_ref):
    a = _dot_t(attn_ref[...], ow_ref[...])
    xr = x_ref[...] + a
    xres_ref[...] = xr
    xn = _rms(xr, nw_ref[...])
    xnb = xn.astype(jnp.bfloat16)
    xn2_ref[...] = xnb
    g = jnp.dot(xnb, wsg_ref[...].astype(jnp.bfloat16),
                preferred_element_type=jnp.float32)
    u = jnp.dot(xnb, wsu_ref[...].astype(jnp.bfloat16),
                preferred_element_type=jnp.float32)
    hs = (g * jax.nn.sigmoid(g) * u).astype(jnp.bfloat16)
    shared_ref[...] = jnp.dot(hs, wsd_ref[...].astype(jnp.bfloat16),
                              preferred_element_type=jnp.float32)


def _router_body(xres_ref, nw_ref, rwt_ref, cw_ref, aux_ref, *, ne, coeff):
    xn = _rms(xres_ref[...], nw_ref[...])
    logits = _dot_t(xn, rwt_ref[...])
    lm = jnp.max(logits, axis=-1, keepdims=True)
    ex = jnp.exp(logits - lm)
    probs = ex / jnp.sum(ex, axis=-1, keepdims=True)
    idx = jax.lax.broadcasted_iota(jnp.int32, probs.shape, 1)
    m1 = jnp.max(probs, axis=-1, keepdims=True)
    i1 = jnp.min(jnp.where(probs == m1, idx, ne), axis=-1, keepdims=True)
    oh1 = (idx == i1)
    p2 = jnp.where(oh1, _NEG, probs)
    m2 = jnp.max(p2, axis=-1, keepdims=True)
    i2 = jnp.min(jnp.where(p2 == m2, idx, ne), axis=-1, keepdims=True)
    oh2 = (idx == i2)
    denom = m1 + m2
    cw_ref[...] = (jnp.where(oh1, m1, 0.0) + jnp.where(oh2, m2, 0.0)) / denom
    frac = jnp.mean((oh1 | oh2).astype(jnp.float32), axis=0, keepdims=True)
    pmean = jnp.mean(probs, axis=0, keepdims=True)
    aux_ref[...] = jnp.sum(frac * pmean).reshape(1, 1) * (coeff * ne)


def _moe_body(xn2_ref, wg_ref, wu_ref, wd_ref, cw_ref, xres_ref, shared_ref,
              o_ref, *, ne):
    e = pl.program_id(0)
    xb = xn2_ref[...]
    g = jnp.dot(xb, wg_ref[0].astype(jnp.bfloat16),
                preferred_element_type=jnp.float32)
    u = jnp.dot(xb, wu_ref[0].astype(jnp.bfloat16),
                preferred_element_type=jnp.float32)
    hh = (g * jax.nn.sigmoid(g) * u).astype(jnp.bfloat16)
    ye = jnp.dot(hh, wd_ref[0].astype(jnp.bfloat16),
                 preferred_element_type=jnp.float32)
    cwb = cw_ref[...]
    lane = jax.lax.broadcasted_iota(jnp.int32, cwb.shape, 1)
    w_col = jnp.sum(jnp.where(lane == e, cwb, 0.0), axis=-1, keepdims=True)
    contrib = w_col * ye

    @pl.when(e == 0)
    def _():
        o_ref[...] = xres_ref[...] + shared_ref[...] + contrib

    @pl.when(e > 0)
    def _():
        o_ref[...] += contrib


def kernel(x, attn_norm_w, qkv_w, out_w, ffn_norm_w, router_w, w_gate, w_up,
           w_down, ws_gate, ws_up, ws_down):
    B, T, D = x.shape
    E, _, F = w_gate.shape
    H = 16
    hd = D // H
    BT = min(256, T)
    BM = min(512, T)
    x2 = x.reshape(T, D)
    bf = jnp.bfloat16

    anw = attn_norm_w.reshape(1, D)
    fnw = ffn_norm_w.reshape(1, D)

    qkv = pl.pallas_call(
        _qkv_body,
        grid=(T // BT,),
        in_specs=[
            pl.BlockSpec((BT, D), lambda i: (i, 0)),
            pl.BlockSpec((1, D), lambda i: (0, 0)),
            pl.BlockSpec((3 * D, D), lambda i: (0, 0)),
        ],
        out_specs=pl.BlockSpec((BT, 3 * D), lambda i: (i, 0)),
        out_shape=jax.ShapeDtypeStruct((T, 3 * D), jnp.float32),
        compiler_params=pltpu.CompilerParams(
            dimension_semantics=("arbitrary",)),
    )(x2, anw, qkv_w)

    BQ = min(512, T)
    attn = pl.pallas_call(
        functools.partial(_attn_body, bq=BQ, hd=hd, nh=H, d=D),
        grid=(T // BQ,),
        in_specs=[
            pl.BlockSpec((T, 3 * D), lambda i: (0, 0)),
        ],
        out_specs=pl.BlockSpec((BQ, D), lambda i: (i, 0)),
        out_shape=jax.ShapeDtypeStruct((T, D), jnp.float32),
        compiler_params=pltpu.CompilerParams(
            dimension_semantics=("arbitrary",)),
    )(qkv)

    xres, xn2, shared = pl.pallas_call(
        _proj_body,
        grid=(T // BT,),
        in_specs=[
            pl.BlockSpec((BT, D), lambda i: (i, 0)),
            pl.BlockSpec((BT, D), lambda i: (i, 0)),
            pl.BlockSpec((D, D), lambda i: (0, 0)),
            pl.BlockSpec((1, D), lambda i: (0, 0)),
            pl.BlockSpec((D, F), lambda i: (0, 0)),
            pl.BlockSpec((D, F), lambda i: (0, 0)),
            pl.BlockSpec((F, D), lambda i: (0, 0)),
        ],
        out_specs=[
            pl.BlockSpec((BT, D), lambda i: (i, 0)),
            pl.BlockSpec((BT, D), lambda i: (i, 0)),
            pl.BlockSpec((BT, D), lambda i: (i, 0)),
        ],
        out_shape=[
            jax.ShapeDtypeStruct((T, D), jnp.float32),
            jax.ShapeDtypeStruct((T, D), bf),
            jax.ShapeDtypeStruct((T, D), jnp.float32),
        ],
        compiler_params=pltpu.CompilerParams(
            dimension_semantics=("arbitrary",)),
    )(attn, x2, out_w, fnw, ws_gate, ws_up, ws_down)

    cw, aux = pl.pallas_call(
        functools.partial(_router_body, ne=E, coeff=0.01),
        grid=(1,),
        in_specs=[
            pl.BlockSpec((T, D), lambda i: (0, 0)),
            pl.BlockSpec((1, D), lambda i: (0, 0)),
            pl.BlockSpec((E, D), lambda i: (0, 0)),
        ],
        out_specs=[
            pl.BlockSpec((T, E), lambda i: (0, 0)),
            pl.BlockSpec((1, 1), lambda i: (0, 0)),
        ],
        out_shape=[
            jax.ShapeDtypeStruct((T, E), jnp.float32),
            jax.ShapeDtypeStruct((1, 1), jnp.float32),
        ],
    )(xres, fnw, router_w)

    y = pl.pallas_call(
        functools.partial(_moe_body, ne=E),
        grid=(E,),
        in_specs=[
            pl.BlockSpec((T, D), lambda e: (0, 0)),
            pl.BlockSpec((1, D, F), lambda e: (e, 0, 0)),
            pl.BlockSpec((1, D, F), lambda e: (e, 0, 0)),
            pl.BlockSpec((1, F, D), lambda e: (e, 0, 0)),
            pl.BlockSpec((T, E), lambda e: (0, 0)),
            pl.BlockSpec((T, D), lambda e: (0, 0)),
            pl.BlockSpec((T, D), lambda e: (0, 0)),
        ],
        out_specs=pl.BlockSpec((T, D), lambda e: (0, 0)),
        out_shape=jax.ShapeDtypeStruct((T, D), jnp.float32),
        compiler_params=pltpu.CompilerParams(
            dimension_semantics=("arbitrary",)),
    )(xn2, w_gate, w_up, w_down, cw, xres, shared)

    return (y.reshape(B, T, D), aux[0, 0])
